# Initial kernel scaffold; baseline (speedup 1.0000x reference)
import jax
import jax.numpy as jnp
from jax.experimental import pallas as pl


def kernel(src, ln1_g, ln1_b, ln2_g, ln2_b, Wq, Wk, Wv, Wo, rel_bias, expert_sel, keys, values):
    raise NotImplementedError("write your pallas kernel here")


# speedup vs baseline: 40.5467x; 40.5467x over previous
import functools

import jax
import jax.numpy as jnp
import numpy as np
from jax import lax
from jax.experimental import pallas as pl
from jax.experimental.pallas import tpu as pltpu

S, D, H, E, F = 2048, 768, 12, 64, 64
DH = D // H
NB = S // 128
ND = 2 * NB - 1

_INTERPRET = False


def _pc(body, grid, in_specs, out_specs, out_shape, scratch_shapes=()):
    return pl.pallas_call(
        body,
        grid=grid,
        in_specs=in_specs,
        out_specs=out_specs,
        out_shape=out_shape,
        scratch_shapes=list(scratch_shapes),
        interpret=_INTERPRET,
    )


def _ln(x, g, b):
    m = jnp.mean(x, axis=-1, keepdims=True)
    v = jnp.mean((x - m) ** 2, axis=-1, keepdims=True)
    return (x - m) * jax.lax.rsqrt(v + 1e-5) * g + b


def _k1_body(src_ref, g_ref, b_ref, wq_ref, wk_ref, wv_ref, q_ref, k_ref, v_ref):
    x2 = _ln(src_ref[...], g_ref[...], b_ref[...]).astype(jnp.bfloat16)
    q = lax.dot(x2, wq_ref[...], preferred_element_type=jnp.float32) * 0.125
    k = lax.dot(x2, wk_ref[...], preferred_element_type=jnp.float32)
    v = lax.dot(x2, wv_ref[...], preferred_element_type=jnp.float32)
    qb, kb, vb = q.astype(jnp.bfloat16), k.astype(jnp.bfloat16), v.astype(jnp.bfloat16)
    for h in range(H):
        sl = slice(h * DH, (h + 1) * DH)
        q_ref[h] = qb[:, sl]
        k_ref[h] = kb[:, sl]
        v_ref[h] = vb[:, sl]


def _k1(src, ln1_g, ln1_b, wq, wk, wv):
    spec_w = pl.BlockSpec((D, D), lambda i: (0, 0))
    spec_v = pl.BlockSpec((1, D), lambda i: (0, 0))
    out_spec = pl.BlockSpec((H, 128, DH), lambda i: (0, i, 0))
    return _pc(
        _k1_body,
        grid=(NB,),
        in_specs=[pl.BlockSpec((128, D), lambda i: (i, 0)), spec_v, spec_v,
                  spec_w, spec_w, spec_w],
        out_specs=[out_spec] * 3,
        out_shape=[jax.ShapeDtypeStruct((H, S, DH), jnp.bfloat16)] * 3,
    )(src, ln1_g.reshape(1, D), ln1_b.reshape(1, D),
      wq.astype(jnp.bfloat16), wk.astype(jnp.bfloat16), wv.astype(jnp.bfloat16))


def _k2_body(bl_ref, br_ref, e_ref, out_ref):
    sm = jnp.concatenate([bl_ref[...], br_ref[...]], axis=1).astype(jnp.bfloat16)
    mm = lax.dot(sm, e_ref[...], preferred_element_type=jnp.float32)
    out_ref[0] = mm.astype(jnp.bfloat16)


def _k2(rel_bias):
    rb = jnp.pad(rel_bias, ((0, 0), (0, 4096 - (2 * S - 1))))
    ab = np.arange(128 * 128)
    a, b = ab // 128, ab % 128
    c = np.arange(256)
    e_mat = (c[:, None] == (b - a + 127)[None, :]).astype(np.float32)
    e_mat = jnp.asarray(e_mat, dtype=jnp.bfloat16)
    t3 = _pc(
        _k2_body,
        grid=(ND,),
        in_specs=[pl.BlockSpec((H, 128), lambda d: (0, d)),
                  pl.BlockSpec((H, 128), lambda d: (0, d + 1)),
                  pl.BlockSpec((256, 128 * 128), lambda d: (0, 0))],
        out_specs=pl.BlockSpec((1, H, 128 * 128), lambda d: (d, 0, 0)),
        out_shape=jax.ShapeDtypeStruct((ND, H, 128 * 128), jnp.bfloat16),
    )(rb, rb, e_mat)
    return t3.reshape(ND, H, 128, 128)


def _k3_body(q_ref, k_ref, v_ref, t_ref, o_ref):
    h = pl.program_id(0)
    i = pl.program_id(1)
    q = q_ref[0]
    k = k_ref[0]
    s = lax.dot_general(q, k, (((1,), (1,)), ((), ())),
                        preferred_element_type=jnp.float32)
    patt = jnp.concatenate(
        [t_ref[j - i + (NB - 1), h].astype(jnp.float32) for j in range(NB)], axis=1)
    s = s + patt
    m = jnp.max(s, axis=1, keepdims=True)
    p = jnp.exp(s - m)
    l = jnp.sum(p, axis=1, keepdims=True)
    att = (p / l).astype(jnp.bfloat16)
    o = lax.dot(att, v_ref[0], preferred_element_type=jnp.float32)
    o_ref[0] = o.astype(jnp.bfloat16)


def _k3(q, k, v, t4):
    return _pc(
        _k3_body,
        grid=(H, NB),
        in_specs=[pl.BlockSpec((1, 128, DH), lambda h, i: (h, i, 0)),
                  pl.BlockSpec((1, S, DH), lambda h, i: (h, 0, 0)),
                  pl.BlockSpec((1, S, DH), lambda h, i: (h, 0, 0)),
                  pl.BlockSpec((ND, H, 128, 128), lambda h, i: (0, 0, 0, 0))],
        out_specs=pl.BlockSpec((1, 128, DH), lambda h, i: (h, i, 0)),
        out_shape=jax.ShapeDtypeStruct((H, S, DH), jnp.bfloat16),
    )(q, k, v, t4)


def _k4_body(att_ref, wo_ref, src_ref, out_ref):
    cat = jnp.concatenate([att_ref[h] for h in range(H)], axis=1)
    o = lax.dot(cat, wo_ref[...], preferred_element_type=jnp.float32)
    out_ref[...] = src_ref[...] + o


def _k4(att, wo, src):
    return _pc(
        _k4_body,
        grid=(NB,),
        in_specs=[pl.BlockSpec((H, 128, DH), lambda i: (0, i, 0)),
                  pl.BlockSpec((D, D), lambda i: (0, 0)),
                  pl.BlockSpec((128, D), lambda i: (i, 0))],
        out_specs=pl.BlockSpec((128, D), lambda i: (i, 0)),
        out_shape=jax.ShapeDtypeStruct((S, D), jnp.float32),
    )(att, wo.astype(jnp.bfloat16), src)


def _k5_body(src_ref, g_ref, b_ref, es_ref, x3_ref, gd_ref):
    x3 = _ln(src_ref[...], g_ref[...], b_ref[...])
    x3_ref[...] = x3
    logits = lax.dot(x3, es_ref[...], preferred_element_type=jnp.float32)
    sel = jax.nn.sigmoid(logits)
    iota = lax.broadcasted_iota(jnp.int32, sel.shape, 1)
    m1 = jnp.max(sel, axis=1, keepdims=True)
    i1 = jnp.min(jnp.where(sel == m1, iota, E), axis=1, keepdims=True)
    masked = jnp.where(iota == i1, -1.0, sel)
    m2 = jnp.max(masked, axis=1, keepdims=True)
    i2 = jnp.min(jnp.where(masked == m2, iota, E), axis=1, keepdims=True)
    gd = jnp.where(iota == i1, m1, 0.0) + jnp.where(iota == i2, m2, 0.0)
    gd_ref[...] = gd


def _k5(src2, ln2_g, ln2_b, expert_sel):
    spec_v = pl.BlockSpec((1, D), lambda i: (0, 0))
    return _pc(
        _k5_body,
        grid=(NB,),
        in_specs=[pl.BlockSpec((128, D), lambda i: (i, 0)), spec_v, spec_v,
                  pl.BlockSpec((D, E), lambda i: (0, 0))],
        out_specs=[pl.BlockSpec((128, D), lambda i: (i, 0)),
                   pl.BlockSpec((128, E), lambda i: (i, 0))],
        out_shape=[jax.ShapeDtypeStruct((S, D), jnp.float32),
                   jax.ShapeDtypeStruct((S, E), jnp.float32)],
    )(src2, ln2_g.reshape(1, D), ln2_b.reshape(1, D), expert_sel)


def _k6_body(x3_ref, gd_ref, w1_ref, w2_ref, r_ref, src_ref, out_ref):
    x3 = x3_ref[...].astype(jnp.bfloat16)
    hid = jax.nn.relu(lax.dot(x3, w1_ref[...], preferred_element_type=jnp.float32))
    grep = lax.dot(gd_ref[...].astype(jnp.bfloat16), r_ref[...],
                   preferred_element_type=jnp.float32)
    y = lax.dot((hid * grep).astype(jnp.bfloat16), w2_ref[...],
                preferred_element_type=jnp.float32)
    out_ref[...] = src_ref[...] + y


def _k6_dense(x3, gd, keys, values, src2):
    w1 = keys.transpose(1, 0, 2).reshape(D, E * F).astype(jnp.bfloat16)
    w2 = values.reshape(E * F, D).astype(jnp.bfloat16)
    r = np.zeros((E, E * F), np.float32)
    r[np.arange(E * F) // F, np.arange(E * F)] = 1.0
    r = jnp.asarray(r, dtype=jnp.bfloat16)
    return _pc(
        _k6_body,
        grid=(NB,),
        in_specs=[pl.BlockSpec((128, D), lambda i: (i, 0)),
                  pl.BlockSpec((128, E), lambda i: (i, 0)),
                  pl.BlockSpec((D, E * F), lambda i: (0, 0)),
                  pl.BlockSpec((E * F, D), lambda i: (0, 0)),
                  pl.BlockSpec((E, E * F), lambda i: (0, 0)),
                  pl.BlockSpec((128, D), lambda i: (i, 0))],
        out_specs=pl.BlockSpec((128, D), lambda i: (i, 0)),
        out_shape=jax.ShapeDtypeStruct((S, D), jnp.float32),
    )(x3, gd, w1, w2, r, src2)


def kernel(src, ln1_g, ln1_b, ln2_g, ln2_b, Wq, Wk, Wv, Wo, rel_bias,
           expert_sel, keys, values):
    src2d = src.reshape(S, D)
    q, k, v = _k1(src2d, ln1_g, ln1_b, Wq, Wk, Wv)
    t4 = _k2(rel_bias)
    att = _k3(q, k, v, t4)
    src2 = _k4(att, Wo, src2d)
    x3, gd = _k5(src2, ln2_g, ln2_b, expert_sel)
    out = _k6_dense(x3, gd, keys, values, src2)
    return out.reshape(1, S, D)
